# baseline (speedup 1.0000x reference)
import functools

import jax
import jax.numpy as jnp
from jax import lax
from jax.experimental import pallas as pl
from jax.experimental.pallas import tpu as pltpu
from jax.experimental.pallas import tpu_sc as plsc

D = 128
L = 200
LANES = 16
GROUPS = D // LANES
NBUF = 4
C = 128
SPLITS = (64, 64)


@functools.lru_cache(maxsize=None)
def _build(B, V):
    NW = 32
    P = (B * L) // NW
    NCH = P // C
    assert NCH % NBUF == 0 and P % C == 0 and sum(SPLITS) == C

    mesh = plsc.VectorSubcoreMesh(core_axis_name="c", subcore_axis_name="s")

    @functools.partial(
        pl.kernel,
        mesh=mesh,
        out_type=jax.ShapeDtypeStruct((B * L, D), jnp.float32),
        scratch_types=[
            pltpu.VMEM((P,), jnp.int32),
            pltpu.VMEM((L, D), jnp.float32),
        ]
        + [pltpu.VMEM((C, D), jnp.float32) for _ in range(NBUF)]
        + [pltpu.SemaphoreType.DMA for _ in range(2 * NBUF)],
    )
    def emb_pe(x_hbm, table_hbm, pe_hbm, out_hbm, idx_v, pe_v, *rest):
        bufs = rest[:NBUF]
        gsems = rest[NBUF:2 * NBUF]
        ssems = rest[2 * NBUF:]

        wid = lax.axis_index("s") * 2 + lax.axis_index("c")
        p0 = wid * P
        pltpu.sync_copy(pe_hbm, pe_v)
        pltpu.sync_copy(x_hbm.at[pl.ds(p0, P)], idx_v)

        def gather(j, b):
            return pltpu.make_async_copy(
                table_hbm.at[idx_v.at[pl.ds(j * C, C)]], bufs[b], gsems[b])

        offs = [sum(SPLITS[:k]) for k in range(len(SPLITS))]

        def store_piece(j, b, h):
            off, n = offs[h], SPLITS[h]
            return pltpu.make_async_copy(
                bufs[b].at[pl.ds(off, n)],
                out_hbm.at[pl.ds(p0 + j * C + off, n)], ssems[b])

        def store_wait(b):
            return pltpu.make_async_copy(
                bufs[b], out_hbm.at[pl.ds(p0, C)], ssems[b])

        for k in range(NBUF - 1):
            gather(k, k).start()

        def add_rows(b, po, lo, hi):
            def add_row(r, c):
                for g in range(GROUPS):
                    sl = pl.ds(g * LANES, LANES)
                    plsc.addupdate(bufs[b].at[r, sl], pe_v[po + r, sl])
                return c

            def add_row_wrap(r, c):
                for g in range(GROUPS):
                    sl = pl.ds(g * LANES, LANES)
                    plsc.addupdate(bufs[b].at[r, sl], pe_v[po + r - L, sl])
                return c

            w = lax.clamp(lo, L - po, hi)
            lax.fori_loop(lo, w, add_row, 0)
            lax.fori_loop(w, hi, add_row_wrap, 0)

        def body(j2, carry):
            for b in range(NBUF):
                pb = (b - 1) % NBUF
                j = j2 * NBUF + b

                @pl.when(j > 0)
                def _():
                    store_wait(pb).wait()

                @pl.when(j + NBUF - 1 < NCH)
                def _():
                    gather(j + NBUF - 1, pb).start()

                gather(j, b).wait()
                po = lax.rem(j * C, L)
                for h in range(len(SPLITS)):
                    add_rows(b, po, offs[h], offs[h] + SPLITS[h])
                    store_piece(j, b, h).start()
            return carry

        lax.fori_loop(0, NCH // NBUF, body, 0)
        store_wait((NCH - 1) % NBUF).wait()

    return emb_pe


def kernel(x, table, pe):
    B, Lx = x.shape
    xi = x.reshape(-1).astype(jnp.int32)
    pef = pe.reshape(Lx, D)
    out = _build(B, table.shape[0])(xi, table, pef)
    return out.reshape(B, Lx, D)


# speedup vs baseline: 1.0015x; 1.0015x over previous
import functools

import jax
import jax.numpy as jnp
from jax import lax
from jax.experimental import pallas as pl
from jax.experimental.pallas import tpu as pltpu
from jax.experimental.pallas import tpu_sc as plsc

D = 128
L = 200
LANES = 16
GROUPS = D // LANES
NBUF = 2
C = 128
SPLITS = (64, 64)


@functools.lru_cache(maxsize=None)
def _build(B, V):
    NW = 32
    P = (B * L) // NW
    NCH = P // C
    assert NCH % NBUF == 0 and P % C == 0 and sum(SPLITS) == C

    mesh = plsc.VectorSubcoreMesh(core_axis_name="c", subcore_axis_name="s")

    @functools.partial(
        pl.kernel,
        mesh=mesh,
        out_type=jax.ShapeDtypeStruct((B * L, D), jnp.float32),
        scratch_types=[
            pltpu.VMEM((P,), jnp.int32),
            pltpu.VMEM((L, D), jnp.float32),
        ]
        + [pltpu.VMEM((C, D), jnp.float32) for _ in range(NBUF)]
        + [pltpu.SemaphoreType.DMA for _ in range(2 * NBUF)],
    )
    def emb_pe(x_hbm, table_hbm, pe_hbm, out_hbm, idx_v, pe_v, *rest):
        bufs = rest[:NBUF]
        gsems = rest[NBUF:2 * NBUF]
        ssems = rest[2 * NBUF:]

        wid = lax.axis_index("s") * 2 + lax.axis_index("c")
        p0 = wid * P
        pltpu.sync_copy(pe_hbm, pe_v)
        pltpu.sync_copy(x_hbm.at[pl.ds(p0, P)], idx_v)

        def gather(j, b):
            return pltpu.make_async_copy(
                table_hbm.at[idx_v.at[pl.ds(j * C, C)]], bufs[b], gsems[b])

        offs = [sum(SPLITS[:k]) for k in range(len(SPLITS))]

        def store_piece(j, b, h):
            off, n = offs[h], SPLITS[h]
            return pltpu.make_async_copy(
                bufs[b].at[pl.ds(off, n)],
                out_hbm.at[pl.ds(p0 + j * C + off, n)], ssems[b])

        def store_wait(b):
            return pltpu.make_async_copy(
                bufs[b], out_hbm.at[pl.ds(p0, C)], ssems[b])

        for k in range(NBUF - 1):
            gather(k, k).start()

        def add_rows(b, po, lo, hi):
            def add_row(r, c):
                for g in range(GROUPS):
                    sl = pl.ds(g * LANES, LANES)
                    plsc.addupdate(bufs[b].at[r, sl], pe_v[po + r, sl])
                return c

            def add_row_wrap(r, c):
                for g in range(GROUPS):
                    sl = pl.ds(g * LANES, LANES)
                    plsc.addupdate(bufs[b].at[r, sl], pe_v[po + r - L, sl])
                return c

            w = lax.clamp(lo, L - po, hi)
            lax.fori_loop(lo, w, add_row, 0)
            lax.fori_loop(w, hi, add_row_wrap, 0)

        def body(j2, carry):
            for b in range(NBUF):
                pb = (b - 1) % NBUF
                j = j2 * NBUF + b

                @pl.when(j > 0)
                def _():
                    store_wait(pb).wait()

                @pl.when(j + NBUF - 1 < NCH)
                def _():
                    gather(j + NBUF - 1, pb).start()

                gather(j, b).wait()
                po = lax.rem(j * C, L)
                for h in range(len(SPLITS)):
                    add_rows(b, po, offs[h], offs[h] + SPLITS[h])
                    store_piece(j, b, h).start()
            return carry

        lax.fori_loop(0, NCH // NBUF, body, 0)
        store_wait((NCH - 1) % NBUF).wait()

    return emb_pe


def kernel(x, table, pe):
    B, Lx = x.shape
    xi = x.reshape(-1).astype(jnp.int32)
    pef = pe.reshape(Lx, D)
    out = _build(B, table.shape[0])(xi, table, pef)
    return out.reshape(B, Lx, D)


# speedup vs baseline: 2.7357x; 2.7316x over previous
import functools

import jax
import jax.numpy as jnp
from jax import lax
from jax.experimental import pallas as pl
from jax.experimental.pallas import tpu as pltpu
from jax.experimental.pallas import tpu_sc as plsc

D = 128
L = 200
LANES = 16
GROUPS = D // LANES
NBUF = 2
C = 200
SPLITS = (40, 32, 32, 32, 32, 32)


@functools.lru_cache(maxsize=None)
def _build(B, V):
    NW = 32
    P = (B * L) // NW
    NCH = P // C
    assert NCH % NBUF == 0 and P % C == 0 and L == C

    mesh = plsc.VectorSubcoreMesh(core_axis_name="c", subcore_axis_name="s")

    @functools.partial(
        pl.kernel,
        mesh=mesh,
        out_type=jax.ShapeDtypeStruct((B * L, D), jnp.float32),
        scratch_types=[
            pltpu.VMEM((P,), jnp.int32),
            pltpu.VMEM((L, D), jnp.float32),
        ]
        + [pltpu.VMEM((C, D), jnp.float32) for _ in range(NBUF)]
        + [pltpu.SemaphoreType.DMA for _ in range(2 * NBUF)],
    )
    def emb_pe(x_hbm, table_hbm, pe_hbm, out_hbm, idx_v, pe_v, *rest):
        bufs = rest[:NBUF]
        gsems = rest[NBUF:2 * NBUF]
        ssems = rest[2 * NBUF:]

        wid = lax.axis_index("s") * 2 + lax.axis_index("c")
        p0 = wid * P
        pltpu.sync_copy(pe_hbm, pe_v)
        pltpu.sync_copy(x_hbm.at[pl.ds(p0, P)], idx_v)

        def gather(j, b):
            return pltpu.make_async_copy(
                table_hbm.at[idx_v.at[pl.ds(j * C, C)]], bufs[b], gsems[b])

        offs = [sum(SPLITS[:k]) for k in range(len(SPLITS))]

        def store_piece(j, b, h):
            off, n = offs[h], SPLITS[h]
            return pltpu.make_async_copy(
                bufs[b].at[pl.ds(off, n)],
                out_hbm.at[pl.ds(p0 + j * C + off, n)], ssems[b])

        def store_wait(b):
            return pltpu.make_async_copy(
                bufs[b], out_hbm.at[pl.ds(p0, C)], ssems[b])

        for k in range(NBUF - 1):
            gather(k, k).start()

        def add_rows(b, lo, hi):
            def add_row(r, c):
                for g in range(GROUPS):
                    sl = pl.ds(g * LANES, LANES)
                    plsc.addupdate(bufs[b].at[r, sl], pe_v[r, sl])
                return c

            lax.fori_loop(lo, hi, add_row, 0)

        def body(j2, carry):
            for b in range(NBUF):
                pb = (b - 1) % NBUF
                j = j2 * NBUF + b

                @pl.when(j > 0)
                def _():
                    store_wait(pb).wait()

                @pl.when(j + NBUF - 1 < NCH)
                def _():
                    gather(j + NBUF - 1, pb).start()

                gather(j, b).wait()
                for h in range(len(SPLITS)):
                    add_rows(b, offs[h], offs[h] + SPLITS[h])
                    store_piece(j, b, h).start()
            return carry

        lax.fori_loop(0, NCH // NBUF, body, 0)
        store_wait((NCH - 1) % NBUF).wait()

    return emb_pe


def kernel(x, table, pe):
    B, Lx = x.shape
    xi = x.reshape(-1).astype(jnp.int32)
    pef = pe.reshape(Lx, D)
    out = _build(B, table.shape[0])(xi, table, pef)
    return out.reshape(B, Lx, D)


# speedup vs baseline: 2.7545x; 1.0069x over previous
import functools

import jax
import jax.numpy as jnp
from jax import lax
from jax.experimental import pallas as pl
from jax.experimental.pallas import tpu as pltpu
from jax.experimental.pallas import tpu_sc as plsc

D = 128
L = 200
LANES = 16
GROUPS = D // LANES
NBUF = 2
C = 200
SPLITS = (40, 32, 32, 32, 32, 32)


@functools.lru_cache(maxsize=None)
def _build(B, V):
    NW = 32
    P = (B * L) // NW
    NCH = P // C
    assert NCH % NBUF == 0 and P % C == 0 and L == C

    mesh = plsc.VectorSubcoreMesh(core_axis_name="c", subcore_axis_name="s")

    @functools.partial(
        pl.kernel,
        mesh=mesh,
        out_type=jax.ShapeDtypeStruct((B * L, D), jnp.float32),
        scratch_types=[
            pltpu.VMEM((P,), jnp.int32),
            pltpu.VMEM((L, D), jnp.float32),
        ]
        + [pltpu.VMEM((C, D), jnp.float32) for _ in range(NBUF)]
        + [pltpu.SemaphoreType.DMA for _ in range(2 * NBUF + 2)],
    )
    def emb_pe(x_hbm, table_hbm, pe_hbm, out_hbm, idx_v, pe_v, *rest):
        bufs = rest[:NBUF]
        gsems = rest[NBUF:2 * NBUF]
        ssems = rest[2 * NBUF:3 * NBUF]
        isem, psem = rest[3 * NBUF], rest[3 * NBUF + 1]

        wid = lax.axis_index("s") * 2 + lax.axis_index("c")
        p0 = wid * P
        idx_cp = pltpu.make_async_copy(x_hbm.at[pl.ds(p0, P)], idx_v, isem)
        pe_cp = pltpu.make_async_copy(pe_hbm, pe_v, psem)
        idx_cp.start()
        pe_cp.start()
        idx_cp.wait()

        def gather(j, b):
            return pltpu.make_async_copy(
                table_hbm.at[idx_v.at[pl.ds(j * C, C)]], bufs[b], gsems[b])

        offs = [sum(SPLITS[:k]) for k in range(len(SPLITS))]

        def store_piece(j, b, h):
            off, n = offs[h], SPLITS[h]
            return pltpu.make_async_copy(
                bufs[b].at[pl.ds(off, n)],
                out_hbm.at[pl.ds(p0 + j * C + off, n)], ssems[b])

        def store_wait(b):
            return pltpu.make_async_copy(
                bufs[b], out_hbm.at[pl.ds(p0, C)], ssems[b])

        for k in range(NBUF - 1):
            gather(k, k).start()
        pe_cp.wait()

        def add_rows(b, lo, hi):
            def add_row(r, c):
                for g in range(GROUPS):
                    sl = pl.ds(g * LANES, LANES)
                    plsc.addupdate(bufs[b].at[r, sl], pe_v[r, sl])
                return c

            lax.fori_loop(lo, hi, add_row, 0)

        def body(j2, carry):
            for b in range(NBUF):
                pb = (b - 1) % NBUF
                j = j2 * NBUF + b

                @pl.when(j > 0)
                def _():
                    store_wait(pb).wait()

                @pl.when(j + NBUF - 1 < NCH)
                def _():
                    gather(j + NBUF - 1, pb).start()

                gather(j, b).wait()
                for h in range(len(SPLITS)):
                    add_rows(b, offs[h], offs[h] + SPLITS[h])
                    store_piece(j, b, h).start()
            return carry

        lax.fori_loop(0, NCH // NBUF, body, 0)
        store_wait((NCH - 1) % NBUF).wait()

    return emb_pe


def kernel(x, table, pe):
    B, Lx = x.shape
    xi = x.reshape(-1).astype(jnp.int32)
    pef = pe.reshape(Lx, D)
    out = _build(B, table.shape[0])(xi, table, pef)
    return out.reshape(B, Lx, D)
